# Initial kernel scaffold; baseline (speedup 1.0000x reference)
import jax
import jax.numpy as jnp
from jax.experimental import pallas as pl


def kernel(x, W, bias):
    raise NotImplementedError("write your pallas kernel here")


# speedup vs baseline: 1.4637x; 1.4637x over previous
import functools

import jax
import jax.numpy as jnp
from jax import lax
from jax.experimental import pallas as pl
from jax.experimental.pallas import tpu as pltpu
from jax.experimental.pallas import tpu_sc as plsc

_BATCH = 16384
_FIELDS = 26
_LANES = 16
_NUM_CORES = 2
_NUM_SUBCORES = 16
_NUM_WORKERS = _NUM_CORES * _NUM_SUBCORES
_ROWS_W = _BATCH // _NUM_WORKERS
_CHUNK = 128
_NCHUNKS = _ROWS_W // _CHUNK
_GROUPS = _ROWS_W // _LANES


def _make_kernel():
  mesh = plsc.VectorSubcoreMesh(core_axis_name="c", subcore_axis_name="s")

  @functools.partial(
      pl.kernel,
      mesh=mesh,
      out_type=jax.ShapeDtypeStruct((_BATCH,), jnp.float32),
      scratch_types=[
          pltpu.VMEM((_FIELDS, _ROWS_W), jnp.int32),
          pltpu.VMEM((_FIELDS, _ROWS_W), jnp.float32),
          pltpu.VMEM((_ROWS_W,), jnp.float32),
          pltpu.VMEM((_LANES,), jnp.float32),
          pltpu.SemaphoreType.DMA,
          pltpu.SemaphoreType.DMA,
      ],
  )
  def body(x_hbm, w_hbm, b_hbm, out_hbm, idx_v, vals_v, acc_v, bias_v,
           sem_i, sem_g):
    wid = lax.axis_index("s") * _NUM_CORES + lax.axis_index("c")
    base = wid * _ROWS_W
    pltpu.sync_copy(b_hbm, bias_v)

    def fire_idx(f, carry):
      pltpu.async_copy(
          x_hbm.at[pl.ds(f * _BATCH + base, _ROWS_W)], idx_v.at[f], sem_i)
      return carry

    lax.fori_loop(0, _FIELDS, fire_idx, None)

    def drain_idx(f, carry):
      pltpu.make_async_copy(
          x_hbm.at[pl.ds(0, _ROWS_W)], idx_v.at[f], sem_i).wait()
      return carry

    lax.fori_loop(0, _FIELDS, drain_idx, None)

    def fire_gather(f, carry):
      for r in range(_NCHUNKS):
        sl = pl.ds(r * _CHUNK, _CHUNK)
        pltpu.async_copy(w_hbm.at[idx_v.at[f, sl]], vals_v.at[f, sl], sem_g)
      return carry

    lax.fori_loop(0, _FIELDS, fire_gather, None)

    def drain_gather(f, carry):
      pltpu.make_async_copy(
          w_hbm.at[pl.ds(0, _ROWS_W)], vals_v.at[f], sem_g).wait()
      return carry

    lax.fori_loop(0, _FIELDS, drain_gather, None)

    bvec = bias_v[...]

    def accumulate(g, carry):
      sl = pl.ds(g * _LANES, _LANES)
      acc = bvec
      for f in range(_FIELDS):
        acc = acc + vals_v[f, sl]
      acc_v[sl] = acc
      return carry

    lax.fori_loop(0, _GROUPS, accumulate, None)
    pltpu.sync_copy(acc_v, out_hbm.at[pl.ds(base, _ROWS_W)])

  return body


_kernel_fn = _make_kernel()


def kernel(x, W, bias):
  xt = jnp.transpose(x).reshape(-1)
  wf = W.reshape(-1)
  b16 = jnp.broadcast_to(bias.astype(jnp.float32), (_LANES,))
  out = _kernel_fn(xt, wf, b16)
  return out.reshape(_BATCH, 1)
